# baseline (speedup 1.0000x reference)
import functools

import jax
import jax.numpy as jnp
from jax import lax
from jax.experimental import pallas as pl
from jax.experimental.pallas import tpu as pltpu
from jax.experimental.pallas import tpu_sc as plsc

_NE = 8
_D = 1024
_F = 2048
_T = 2048
_P = 2 * _T
_BLK = 256
_NB = _P // _BLK + _NE
_NBB = _NB * _BLK
_CH = 128
_NCH = _P // _CH

_TCB = 256


def _route_kernel(x_ref, gw_ref, gb_ref,
                  pos0_ref, pos1_ref, w0_ref, w1_ref, be_ref, bv_ref,
                  loss_ref, oh_ref, rk_ref):
    x = x_ref[...]
    logits = lax.dot_general(
        x, gw_ref[...], (((1,), (0,)), ((), ())),
        preferred_element_type=jnp.float32) + gb_ref[...]
    m = jnp.max(logits, axis=1, keepdims=True)
    ex = jnp.exp(logits - m)
    p = ex / jnp.sum(ex, axis=1, keepdims=True)

    idx = lax.broadcasted_iota(jnp.int32, (_T, _NE), 1)
    m1 = jnp.max(p, axis=1, keepdims=True)
    e1 = jnp.min(jnp.where(p == m1, idx, _NE), axis=1, keepdims=True)
    p2 = jnp.where(idx == e1, -jnp.inf, p)
    m2 = jnp.max(p2, axis=1, keepdims=True)
    e2 = jnp.min(jnp.where(p2 == m2, idx, _NE), axis=1, keepdims=True)

    s = m1 + m2
    w0_ref[...] = m1 / s
    w1_ref[...] = m2 / s

    oh1 = (idx == e1).astype(jnp.float32)
    oh2 = (idx == e2).astype(jnp.float32)
    oh_ref[0:_T, :] = oh1
    oh_ref[_T:_P, :] = oh2

    ri = lax.broadcasted_iota(jnp.int32, (_CH, _CH), 0)
    ci = lax.broadcasted_iota(jnp.int32, (_CH, _CH), 1)
    trils = (ri > ci).astype(jnp.float32)

    def body(c, carry):
        st = pl.multiple_of(c * _CH, _CH)
        chunk = oh_ref[pl.ds(st, _CH), :]
        rm = lax.dot_general(
            trils, chunk, (((1,), (0,)), ((), ())),
            preferred_element_type=jnp.float32) + carry
        rk_ref[pl.ds(st, _CH), :] = rm * chunk
        return carry + jnp.sum(chunk, axis=0, keepdims=True)

    counts = lax.fori_loop(0, _NCH, body, jnp.zeros((1, _NE), jnp.float32))

    nblk = jnp.floor((counts + (_BLK - 1)) * (1.0 / _BLK))
    ei = lax.broadcasted_iota(jnp.int32, (_NE, _NE), 0)
    ej = lax.broadcasted_iota(jnp.int32, (_NE, _NE), 1)
    ustri = (ei < ej).astype(jnp.float32)
    ecum = lax.dot_general(
        nblk, ustri, (((1,), (0,)), ((), ())),
        preferred_element_type=jnp.float32)
    pstart = ecum * float(_BLK)

    ohall = oh_ref[...]
    ps_sel = jnp.sum(ohall * pstart, axis=1, keepdims=True)
    rk_sel = jnp.sum(rk_ref[...], axis=1, keepdims=True)
    pos = (ps_sel + rk_sel).astype(jnp.int32)
    pos0_ref[...] = pos[0:_T]
    pos1_ref[...] = pos[_T:_P]

    bi = lax.broadcasted_iota(jnp.int32, (_NB, _NE), 0).astype(jnp.float32)
    bstart = jnp.broadcast_to(ecum, (_NB, _NE))
    be = jnp.sum((bstart <= bi).astype(jnp.int32), axis=1, keepdims=True) - 1
    be_ref[...] = be
    total = jnp.sum(nblk)
    bvi = lax.broadcasted_iota(jnp.int32, (_NB, 1), 0).astype(jnp.float32)
    bv_ref[...] = (bvi < total).astype(jnp.int32)

    psum = jnp.sum(p, axis=0, keepdims=True)
    loss = _NE * jnp.sum(counts * psum) / float(_T * _T)
    loss_ref[...] = jnp.reshape(loss, (1, 1))


def _ffn_kernel(be_ref, bv_ref, xs_ref, W1_ref, b1_ref, W2_ref, b2_ref,
                ys_ref):
    f = pl.program_id(1)

    @pl.when(bv_ref[pl.program_id(0)] == 1)
    def _():
        h = lax.dot_general(
            xs_ref[...], W1_ref[0], (((1,), (0,)), ((), ())),
            preferred_element_type=jnp.float32) + b1_ref[0]
        h = jnp.maximum(h, 0.0)
        y = lax.dot_general(
            h, W2_ref[0], (((1,), (0,)), ((), ())),
            preferred_element_type=jnp.float32)

        @pl.when(f == 0)
        def _():
            ys_ref[...] = y + b2_ref[0]

        @pl.when(f != 0)
        def _():
            ys_ref[...] += y


def _combine_kernel(y0_ref, y1_ref, w0_ref, w1_ref, out_ref):
    out_ref[...] = w0_ref[...] * y0_ref[...] + w1_ref[...] * y1_ref[...]


def _sc_wid():
    info = plsc.get_sparse_core_info()
    return lax.axis_index("s") * info.num_cores + lax.axis_index("c")


def _dispatch_body(x_hbm, pos0_hbm, pos1_hbm, xs_hbm, idx_v, rows_v, sem):
    base = _sc_wid() * (_T // 32)
    n = _T // 32
    pltpu.sync_copy(x_hbm.at[pl.ds(base, n)], rows_v)
    pltpu.sync_copy(pos0_hbm.at[pl.ds(base, n)], idx_v)
    pltpu.async_copy(rows_v, xs_hbm.at[idx_v], sem).wait()
    pltpu.sync_copy(pos1_hbm.at[pl.ds(base, n)], idx_v)
    pltpu.async_copy(rows_v, xs_hbm.at[idx_v], sem).wait()


def _gather_body(ys_hbm, pos0_hbm, pos1_hbm, y0_hbm, y1_hbm,
                 idx_v, rows_v, sem):
    base = _sc_wid() * (_T // 32)
    n = _T // 32
    pltpu.sync_copy(pos0_hbm.at[pl.ds(base, n)], idx_v)
    pltpu.async_copy(ys_hbm.at[idx_v], rows_v, sem).wait()
    pltpu.sync_copy(rows_v, y0_hbm.at[pl.ds(base, n)])
    pltpu.sync_copy(pos1_hbm.at[pl.ds(base, n)], idx_v)
    pltpu.async_copy(ys_hbm.at[idx_v], rows_v, sem).wait()
    pltpu.sync_copy(rows_v, y1_hbm.at[pl.ds(base, n)])


def kernel(inputs, gate_W, gate_b, W1, b1, W2, b2):
    x = inputs.reshape(-1, _D)

    pos0, pos1, w0, w1, be, bv, loss = pl.pallas_call(
        _route_kernel,
        out_shape=[
            jax.ShapeDtypeStruct((_T, 1), jnp.int32),
            jax.ShapeDtypeStruct((_T, 1), jnp.int32),
            jax.ShapeDtypeStruct((_T, 1), jnp.float32),
            jax.ShapeDtypeStruct((_T, 1), jnp.float32),
            jax.ShapeDtypeStruct((_NB, 1), jnp.int32),
            jax.ShapeDtypeStruct((_NB, 1), jnp.int32),
            jax.ShapeDtypeStruct((1, 1), jnp.float32),
        ],
        scratch_shapes=[
            pltpu.VMEM((_P, _NE), jnp.float32),
            pltpu.VMEM((_P, _NE), jnp.float32),
        ],
    )(x, gate_W, gate_b.reshape(1, _NE))

    pos0f = pos0.reshape(_T)
    pos1f = pos1.reshape(_T)

    mesh = plsc.VectorSubcoreMesh(core_axis_name="c", subcore_axis_name="s")
    n = _T // 32
    xs = pl.kernel(
        _dispatch_body,
        mesh=mesh,
        out_type=jax.ShapeDtypeStruct((_NBB, _D), jnp.float32),
        scratch_types=[
            pltpu.VMEM((n,), jnp.int32),
            pltpu.VMEM((n, _D), jnp.float32),
            pltpu.SemaphoreType.DMA,
        ],
    )(x, pos0f, pos1f)

    fh = _F // 2
    ys = pl.pallas_call(
        _ffn_kernel,
        grid_spec=pltpu.PrefetchScalarGridSpec(
            num_scalar_prefetch=2,
            grid=(_NB, 2),
            in_specs=[
                pl.BlockSpec((_BLK, _D), lambda b, f, be_r, bv_r: (b, 0)),
                pl.BlockSpec((1, _D, fh), lambda b, f, be_r, bv_r: (be_r[b], 0, f)),
                pl.BlockSpec((1, 1, fh), lambda b, f, be_r, bv_r: (be_r[b], 0, f)),
                pl.BlockSpec((1, fh, _D), lambda b, f, be_r, bv_r: (be_r[b], f, 0)),
                pl.BlockSpec((1, 1, _D), lambda b, f, be_r, bv_r: (be_r[b], 0, 0)),
            ],
            out_specs=pl.BlockSpec((_BLK, _D), lambda b, f, be_r, bv_r: (b, 0)),
        ),
        out_shape=jax.ShapeDtypeStruct((_NBB, _D), jnp.float32),
        compiler_params=pltpu.CompilerParams(
            dimension_semantics=("arbitrary", "arbitrary"),
        ),
    )(be.reshape(_NB), bv.reshape(_NB), xs, W1,
      b1.reshape(_NE, 1, _F), W2, b2.reshape(_NE, 1, _D))

    y0, y1 = pl.kernel(
        _gather_body,
        mesh=mesh,
        out_type=[
            jax.ShapeDtypeStruct((_T, _D), jnp.float32),
            jax.ShapeDtypeStruct((_T, _D), jnp.float32),
        ],
        scratch_types=[
            pltpu.VMEM((n,), jnp.int32),
            pltpu.VMEM((n, _D), jnp.float32),
            pltpu.SemaphoreType.DMA,
        ],
    )(ys, pos0f, pos1f)

    out = pl.pallas_call(
        _combine_kernel,
        grid=(_T // _TCB,),
        in_specs=[
            pl.BlockSpec((_TCB, _D), lambda t: (t, 0)),
            pl.BlockSpec((_TCB, _D), lambda t: (t, 0)),
            pl.BlockSpec((_TCB, 1), lambda t: (t, 0)),
            pl.BlockSpec((_TCB, 1), lambda t: (t, 0)),
        ],
        out_specs=pl.BlockSpec((_TCB, _D), lambda t: (t, 0)),
        out_shape=jax.ShapeDtypeStruct((_T, _D), jnp.float32),
    )(y0, y1, w0, w1)

    return out.reshape(inputs.shape), loss[0, 0]


# speedup vs baseline: 1.2870x; 1.2870x over previous
import functools

import jax
import jax.numpy as jnp
from jax import lax
from jax.experimental import pallas as pl
from jax.experimental.pallas import tpu as pltpu
from jax.experimental.pallas import tpu_sc as plsc

_NE = 8
_D = 1024
_F = 2048
_T = 2048
_P = 2 * _T
_BLK = 256
_NB = _P // _BLK + _NE
_NBB = _NB * _BLK
_CH = 512
_NCH = _P // _CH

_TCB = 256


def _route_kernel(x_ref, gw_ref, gb_ref,
                  pos0_ref, pos1_ref, w0_ref, w1_ref, be_ref, bv_ref,
                  loss_ref, oh_ref, rk_ref):
    x = x_ref[...]
    logits = lax.dot_general(
        x, gw_ref[...], (((1,), (0,)), ((), ())),
        preferred_element_type=jnp.float32) + gb_ref[...]
    m = jnp.max(logits, axis=1, keepdims=True)
    ex = jnp.exp(logits - m)
    p = ex / jnp.sum(ex, axis=1, keepdims=True)

    idx = lax.broadcasted_iota(jnp.int32, (_T, _NE), 1)
    m1 = jnp.max(p, axis=1, keepdims=True)
    e1 = jnp.min(jnp.where(p == m1, idx, _NE), axis=1, keepdims=True)
    p2 = jnp.where(idx == e1, -jnp.inf, p)
    m2 = jnp.max(p2, axis=1, keepdims=True)
    e2 = jnp.min(jnp.where(p2 == m2, idx, _NE), axis=1, keepdims=True)

    s = m1 + m2
    w0_ref[...] = m1 / s
    w1_ref[...] = m2 / s

    oh1 = (idx == e1).astype(jnp.float32)
    oh2 = (idx == e2).astype(jnp.float32)
    oh_ref[0:_T, :] = oh1
    oh_ref[_T:_P, :] = oh2

    ri = lax.broadcasted_iota(jnp.int32, (_CH, _CH), 0)
    ci = lax.broadcasted_iota(jnp.int32, (_CH, _CH), 1)
    trils = (ri > ci).astype(jnp.float32)

    def body(c, carry):
        st = pl.multiple_of(c * _CH, _CH)
        chunk = oh_ref[pl.ds(st, _CH), :]
        rm = lax.dot_general(
            trils, chunk, (((1,), (0,)), ((), ())),
            preferred_element_type=jnp.float32) + carry
        rk_ref[pl.ds(st, _CH), :] = rm * chunk
        return carry + jnp.sum(chunk, axis=0, keepdims=True)

    counts = lax.fori_loop(0, _NCH, body, jnp.zeros((1, _NE), jnp.float32))

    nblk = jnp.floor((counts + (_BLK - 1)) * (1.0 / _BLK))
    ei = lax.broadcasted_iota(jnp.int32, (_NE, _NE), 0)
    ej = lax.broadcasted_iota(jnp.int32, (_NE, _NE), 1)
    ustri = (ei < ej).astype(jnp.float32)
    ecum = lax.dot_general(
        nblk, ustri, (((1,), (0,)), ((), ())),
        preferred_element_type=jnp.float32)
    pstart = ecum * float(_BLK)

    ohall = oh_ref[...]
    ps_sel = jnp.sum(ohall * pstart, axis=1, keepdims=True)
    rk_sel = jnp.sum(rk_ref[...], axis=1, keepdims=True)
    pos = (ps_sel + rk_sel).astype(jnp.int32)
    pos0_ref[...] = pos[0:_T]
    pos1_ref[...] = pos[_T:_P]

    bi = lax.broadcasted_iota(jnp.int32, (_NB, _NE), 0).astype(jnp.float32)
    bstart = jnp.broadcast_to(ecum, (_NB, _NE))
    be = jnp.sum((bstart <= bi).astype(jnp.int32), axis=1, keepdims=True) - 1
    be_ref[...] = be
    total = jnp.sum(nblk)
    bvi = lax.broadcasted_iota(jnp.int32, (_NB, 1), 0).astype(jnp.float32)
    bv_ref[...] = (bvi < total).astype(jnp.int32)

    psum = jnp.sum(p, axis=0, keepdims=True)
    loss = _NE * jnp.sum(counts * psum) / float(_T * _T)
    loss_ref[...] = jnp.reshape(loss, (1, 1))


def _ffn_kernel(be_ref, bv_ref, xs_ref, W1_ref, b1_ref, W2_ref, b2_ref,
                ys_ref):
    @pl.when(bv_ref[pl.program_id(0)] == 1)
    def _():
        h = lax.dot_general(
            xs_ref[...], W1_ref[0], (((1,), (0,)), ((), ())),
            preferred_element_type=jnp.float32) + b1_ref[0]
        h = jnp.maximum(h, 0.0)
        ys_ref[...] = lax.dot_general(
            h, W2_ref[0], (((1,), (0,)), ((), ())),
            preferred_element_type=jnp.float32) + b2_ref[0]


def _combine_kernel(y0_ref, y1_ref, w0_ref, w1_ref, out_ref):
    out_ref[...] = w0_ref[...] * y0_ref[...] + w1_ref[...] * y1_ref[...]


def _sc_wid():
    info = plsc.get_sparse_core_info()
    return lax.axis_index("s") * info.num_cores + lax.axis_index("c")


def _dispatch_body(x_hbm, pos0_hbm, pos1_hbm, xs_hbm, idx_v, rows_v, sem):
    base = _sc_wid() * (_T // 32)
    n = _T // 32
    pltpu.sync_copy(x_hbm.at[pl.ds(base, n)], rows_v)
    pltpu.sync_copy(pos0_hbm.at[pl.ds(base, n)], idx_v)
    pltpu.async_copy(rows_v, xs_hbm.at[idx_v], sem).wait()
    pltpu.sync_copy(pos1_hbm.at[pl.ds(base, n)], idx_v)
    pltpu.async_copy(rows_v, xs_hbm.at[idx_v], sem).wait()


def _gather_body(ys_hbm, pos0_hbm, pos1_hbm, y0_hbm, y1_hbm,
                 idx_v, rows_v, sem):
    base = _sc_wid() * (_T // 32)
    n = _T // 32
    pltpu.sync_copy(pos0_hbm.at[pl.ds(base, n)], idx_v)
    pltpu.async_copy(ys_hbm.at[idx_v], rows_v, sem).wait()
    pltpu.sync_copy(rows_v, y0_hbm.at[pl.ds(base, n)])
    pltpu.sync_copy(pos1_hbm.at[pl.ds(base, n)], idx_v)
    pltpu.async_copy(ys_hbm.at[idx_v], rows_v, sem).wait()
    pltpu.sync_copy(rows_v, y1_hbm.at[pl.ds(base, n)])


def kernel(inputs, gate_W, gate_b, W1, b1, W2, b2):
    x = inputs.reshape(-1, _D)

    pos0, pos1, w0, w1, be, bv, loss = pl.pallas_call(
        _route_kernel,
        out_shape=[
            jax.ShapeDtypeStruct((_T, 1), jnp.int32),
            jax.ShapeDtypeStruct((_T, 1), jnp.int32),
            jax.ShapeDtypeStruct((_T, 1), jnp.float32),
            jax.ShapeDtypeStruct((_T, 1), jnp.float32),
            jax.ShapeDtypeStruct((_NB, 1), jnp.int32),
            jax.ShapeDtypeStruct((_NB, 1), jnp.int32),
            jax.ShapeDtypeStruct((1, 1), jnp.float32),
        ],
        scratch_shapes=[
            pltpu.VMEM((_P, _NE), jnp.float32),
            pltpu.VMEM((_P, _NE), jnp.float32),
        ],
    )(x, gate_W, gate_b.reshape(1, _NE))

    pos0f = pos0.reshape(_T)
    pos1f = pos1.reshape(_T)

    mesh = plsc.VectorSubcoreMesh(core_axis_name="c", subcore_axis_name="s")
    n = _T // 32
    xs = pl.kernel(
        _dispatch_body,
        mesh=mesh,
        out_type=jax.ShapeDtypeStruct((_NBB, _D), jnp.float32),
        scratch_types=[
            pltpu.VMEM((n,), jnp.int32),
            pltpu.VMEM((n, _D), jnp.float32),
            pltpu.SemaphoreType.DMA,
        ],
    )(x, pos0f, pos1f)

    ys = pl.pallas_call(
        _ffn_kernel,
        grid_spec=pltpu.PrefetchScalarGridSpec(
            num_scalar_prefetch=2,
            grid=(_NB,),
            in_specs=[
                pl.BlockSpec((_BLK, _D), lambda b, be_r, bv_r: (b, 0)),
                pl.BlockSpec((1, _D, _F), lambda b, be_r, bv_r: (be_r[b], 0, 0)),
                pl.BlockSpec((1, 1, _F), lambda b, be_r, bv_r: (be_r[b], 0, 0)),
                pl.BlockSpec((1, _F, _D), lambda b, be_r, bv_r: (be_r[b], 0, 0)),
                pl.BlockSpec((1, 1, _D), lambda b, be_r, bv_r: (be_r[b], 0, 0)),
            ],
            out_specs=pl.BlockSpec((_BLK, _D), lambda b, be_r, bv_r: (b, 0)),
        ),
        out_shape=jax.ShapeDtypeStruct((_NBB, _D), jnp.float32),
        compiler_params=pltpu.CompilerParams(
            dimension_semantics=("arbitrary",),
        ),
    )(be.reshape(_NB), bv.reshape(_NB), xs, W1,
      b1.reshape(_NE, 1, _F), W2, b2.reshape(_NE, 1, _D))

    y0, y1 = pl.kernel(
        _gather_body,
        mesh=mesh,
        out_type=[
            jax.ShapeDtypeStruct((_T, _D), jnp.float32),
            jax.ShapeDtypeStruct((_T, _D), jnp.float32),
        ],
        scratch_types=[
            pltpu.VMEM((n,), jnp.int32),
            pltpu.VMEM((n, _D), jnp.float32),
            pltpu.SemaphoreType.DMA,
        ],
    )(ys, pos0f, pos1f)

    out = pl.pallas_call(
        _combine_kernel,
        grid=(_T // _TCB,),
        in_specs=[
            pl.BlockSpec((_TCB, _D), lambda t: (t, 0)),
            pl.BlockSpec((_TCB, _D), lambda t: (t, 0)),
            pl.BlockSpec((_TCB, 1), lambda t: (t, 0)),
            pl.BlockSpec((_TCB, 1), lambda t: (t, 0)),
        ],
        out_specs=pl.BlockSpec((_TCB, _D), lambda t: (t, 0)),
        out_shape=jax.ShapeDtypeStruct((_T, _D), jnp.float32),
    )(y0, y1, w0, w1)

    return out.reshape(inputs.shape), loss[0, 0]


# speedup vs baseline: 1.2925x; 1.0043x over previous
import functools

import jax
import jax.numpy as jnp
from jax import lax
from jax.experimental import pallas as pl
from jax.experimental.pallas import tpu as pltpu
from jax.experimental.pallas import tpu_sc as plsc

_NE = 8
_D = 1024
_F = 2048
_T = 2048
_P = 2 * _T
_BLK = 256
_NB = _P // _BLK + _NE
_NBB = _NB * _BLK
_CH = 512
_NCH = _P // _CH

_TCB = 256


def _route_kernel(x_ref, gw_ref, gb_ref,
                  pos0_ref, pos1_ref, w0_ref, w1_ref, be_ref, bv_ref,
                  loss_ref, oh_ref, rk_ref):
    x = x_ref[...]
    logits = lax.dot_general(
        x, gw_ref[...], (((1,), (0,)), ((), ())),
        preferred_element_type=jnp.float32) + gb_ref[...]
    m = jnp.max(logits, axis=1, keepdims=True)
    ex = jnp.exp(logits - m)
    p = ex / jnp.sum(ex, axis=1, keepdims=True)

    idx = lax.broadcasted_iota(jnp.int32, (_T, _NE), 1)
    m1 = jnp.max(p, axis=1, keepdims=True)
    e1 = jnp.min(jnp.where(p == m1, idx, _NE), axis=1, keepdims=True)
    p2 = jnp.where(idx == e1, -jnp.inf, p)
    m2 = jnp.max(p2, axis=1, keepdims=True)
    e2 = jnp.min(jnp.where(p2 == m2, idx, _NE), axis=1, keepdims=True)

    s = m1 + m2
    w0_ref[...] = jnp.broadcast_to(m1 / s, (_T, 16))
    w1_ref[...] = jnp.broadcast_to(m2 / s, (_T, 16))

    oh1 = (idx == e1).astype(jnp.float32)
    oh2 = (idx == e2).astype(jnp.float32)
    oh_ref[0:_T, :] = oh1
    oh_ref[_T:_P, :] = oh2

    ri = lax.broadcasted_iota(jnp.int32, (_CH, _CH), 0)
    ci = lax.broadcasted_iota(jnp.int32, (_CH, _CH), 1)
    trils = (ri > ci).astype(jnp.float32)

    def body(c, carry):
        st = pl.multiple_of(c * _CH, _CH)
        chunk = oh_ref[pl.ds(st, _CH), :]
        rm = lax.dot_general(
            trils, chunk, (((1,), (0,)), ((), ())),
            preferred_element_type=jnp.float32) + carry
        rk_ref[pl.ds(st, _CH), :] = rm * chunk
        return carry + jnp.sum(chunk, axis=0, keepdims=True)

    counts = lax.fori_loop(0, _NCH, body, jnp.zeros((1, _NE), jnp.float32))

    nblk = jnp.floor((counts + (_BLK - 1)) * (1.0 / _BLK))
    ei = lax.broadcasted_iota(jnp.int32, (_NE, _NE), 0)
    ej = lax.broadcasted_iota(jnp.int32, (_NE, _NE), 1)
    ustri = (ei < ej).astype(jnp.float32)
    ecum = lax.dot_general(
        nblk, ustri, (((1,), (0,)), ((), ())),
        preferred_element_type=jnp.float32)
    pstart = ecum * float(_BLK)

    ohall = oh_ref[...]
    ps_sel = jnp.sum(ohall * pstart, axis=1, keepdims=True)
    rk_sel = jnp.sum(rk_ref[...], axis=1, keepdims=True)
    pos = (ps_sel + rk_sel).astype(jnp.int32)
    pos0_ref[...] = pos[0:_T]
    pos1_ref[...] = pos[_T:_P]

    bi = lax.broadcasted_iota(jnp.int32, (_NB, _NE), 0).astype(jnp.float32)
    bstart = jnp.broadcast_to(ecum, (_NB, _NE))
    be = jnp.sum((bstart <= bi).astype(jnp.int32), axis=1, keepdims=True) - 1
    be_ref[...] = be
    total = jnp.sum(nblk)
    bvi = lax.broadcasted_iota(jnp.int32, (_NB, 1), 0).astype(jnp.float32)
    bv_ref[...] = (bvi < total).astype(jnp.int32)

    psum = jnp.sum(p, axis=0, keepdims=True)
    loss = _NE * jnp.sum(counts * psum) / float(_T * _T)
    loss_ref[...] = jnp.reshape(loss, (1, 1))


def _ffn_kernel(be_ref, bv_ref, xs_ref, W1_ref, b1_ref, W2_ref, b2_ref,
                ys_ref):
    @pl.when(bv_ref[pl.program_id(0)] == 1)
    def _():
        h = lax.dot_general(
            xs_ref[...], W1_ref[0], (((1,), (0,)), ((), ())),
            preferred_element_type=jnp.float32) + b1_ref[0]
        h = jnp.maximum(h, 0.0)
        ys_ref[...] = lax.dot_general(
            h, W2_ref[0], (((1,), (0,)), ((), ())),
            preferred_element_type=jnp.float32) + b2_ref[0]


def _sc_wid():
    info = plsc.get_sparse_core_info()
    return lax.axis_index("s") * info.num_cores + lax.axis_index("c")


def _dispatch_body(x_hbm, pos0_hbm, pos1_hbm, xs_hbm, idx_v, rows_v, sem):
    base = _sc_wid() * (_T // 32)
    n = _T // 32
    pltpu.sync_copy(x_hbm.at[pl.ds(base, n)], rows_v)
    pltpu.sync_copy(pos0_hbm.at[pl.ds(base, n)], idx_v)
    pltpu.async_copy(rows_v, xs_hbm.at[idx_v], sem).wait()
    pltpu.sync_copy(pos1_hbm.at[pl.ds(base, n)], idx_v)
    pltpu.async_copy(rows_v, xs_hbm.at[idx_v], sem).wait()


def _combine_sc_body(ys_hbm, pos0_hbm, pos1_hbm, w0x_hbm, w1x_hbm, out_hbm,
                     idx_v, r0_v, r1_v, w0_v, w1_v, sem):
    nh = _T // 64
    for h in range(2):
        base = _sc_wid() * (_T // 32) + h * nh
        pltpu.sync_copy(pos0_hbm.at[pl.ds(base, nh)], idx_v)
        pltpu.async_copy(ys_hbm.at[idx_v], r0_v, sem).wait()
        pltpu.sync_copy(pos1_hbm.at[pl.ds(base, nh)], idx_v)
        pltpu.async_copy(ys_hbm.at[idx_v], r1_v, sem).wait()
        pltpu.sync_copy(w0x_hbm.at[pl.ds(base, nh)], w0_v)
        pltpu.sync_copy(w1x_hbm.at[pl.ds(base, nh)], w1_v)

        def row(i, _):
            a = w0_v[i]
            b = w1_v[i]
            for j in range(_D // 16):
                r0_v[i, pl.ds(j * 16, 16)] = (
                    a * r0_v[i, pl.ds(j * 16, 16)]
                    + b * r1_v[i, pl.ds(j * 16, 16)])
            return 0

        lax.fori_loop(0, nh, row, 0)
        pltpu.sync_copy(r0_v, out_hbm.at[pl.ds(base, nh)])


def kernel(inputs, gate_W, gate_b, W1, b1, W2, b2):
    x = inputs.reshape(-1, _D)

    pos0, pos1, w0, w1, be, bv, loss = pl.pallas_call(
        _route_kernel,
        out_shape=[
            jax.ShapeDtypeStruct((_T, 1), jnp.int32),
            jax.ShapeDtypeStruct((_T, 1), jnp.int32),
            jax.ShapeDtypeStruct((_T, 16), jnp.float32),
            jax.ShapeDtypeStruct((_T, 16), jnp.float32),
            jax.ShapeDtypeStruct((_NB, 1), jnp.int32),
            jax.ShapeDtypeStruct((_NB, 1), jnp.int32),
            jax.ShapeDtypeStruct((1, 1), jnp.float32),
        ],
        scratch_shapes=[
            pltpu.VMEM((_P, _NE), jnp.float32),
            pltpu.VMEM((_P, _NE), jnp.float32),
        ],
    )(x, gate_W, gate_b.reshape(1, _NE))

    pos0f = pos0.reshape(_T)
    pos1f = pos1.reshape(_T)

    mesh = plsc.VectorSubcoreMesh(core_axis_name="c", subcore_axis_name="s")
    n = _T // 32
    xs = pl.kernel(
        _dispatch_body,
        mesh=mesh,
        out_type=jax.ShapeDtypeStruct((_NBB, _D), jnp.float32),
        scratch_types=[
            pltpu.VMEM((n,), jnp.int32),
            pltpu.VMEM((n, _D), jnp.float32),
            pltpu.SemaphoreType.DMA,
        ],
    )(x, pos0f, pos1f)

    ys = pl.pallas_call(
        _ffn_kernel,
        grid_spec=pltpu.PrefetchScalarGridSpec(
            num_scalar_prefetch=2,
            grid=(_NB,),
            in_specs=[
                pl.BlockSpec((_BLK, _D), lambda b, be_r, bv_r: (b, 0)),
                pl.BlockSpec((1, _D, _F), lambda b, be_r, bv_r: (be_r[b], 0, 0)),
                pl.BlockSpec((1, 1, _F), lambda b, be_r, bv_r: (be_r[b], 0, 0)),
                pl.BlockSpec((1, _F, _D), lambda b, be_r, bv_r: (be_r[b], 0, 0)),
                pl.BlockSpec((1, 1, _D), lambda b, be_r, bv_r: (be_r[b], 0, 0)),
            ],
            out_specs=pl.BlockSpec((_BLK, _D), lambda b, be_r, bv_r: (b, 0)),
        ),
        out_shape=jax.ShapeDtypeStruct((_NBB, _D), jnp.float32),
        compiler_params=pltpu.CompilerParams(
            dimension_semantics=("arbitrary",),
        ),
    )(be.reshape(_NB), bv.reshape(_NB), xs, W1,
      b1.reshape(_NE, 1, _F), W2, b2.reshape(_NE, 1, _D))

    nh = _T // 64
    out = pl.kernel(
        _combine_sc_body,
        mesh=mesh,
        out_type=jax.ShapeDtypeStruct((_T, _D), jnp.float32),
        scratch_types=[
            pltpu.VMEM((nh,), jnp.int32),
            pltpu.VMEM((nh, _D), jnp.float32),
            pltpu.VMEM((nh, _D), jnp.float32),
            pltpu.VMEM((nh, 16), jnp.float32),
            pltpu.VMEM((nh, 16), jnp.float32),
            pltpu.SemaphoreType.DMA,
        ],
    )(ys, pos0f, pos1f, w0, w1)

    return out.reshape(inputs.shape), loss[0, 0]
